# Initial kernel scaffold; baseline (speedup 1.0000x reference)
import jax
import jax.numpy as jnp
from jax.experimental import pallas as pl


def kernel(spas_item_id, wl_id, wf_loc_id, wf_loc_x, wf_loc_y, spas_table, wl_table, loc_table, W1, b1, W2, b2, W3, b3):
    raise NotImplementedError("write your pallas kernel here")


# speedup vs baseline: 5.4503x; 5.4503x over previous
import functools

import jax
import jax.numpy as jnp
from jax import lax
from jax.experimental import pallas as pl
from jax.experimental.pallas import tpu as pltpu
from jax.experimental.pallas import tpu_sc as plsc

_B = 16384
_H = 128
_NC = 2
_NS = 16
_NW = _NC * _NS
_RPW = _B // _NW
_CHUNK = 128
_NCHUNK = _RPW // _CHUNK


def _sc_gather_body(spas_t, wl_t, loc_t, spas_i, wl_i, loc_i,
                    out_s, out_w, out_l, idx_v, rows_v, sem):
    wid = lax.axis_index("s") * _NC + lax.axis_index("c")
    base = wid * _RPW
    for table, idx_hbm, out_hbm in ((spas_t, spas_i, out_s),
                                    (wl_t, wl_i, out_w),
                                    (loc_t, loc_i, out_l)):
        pltpu.sync_copy(idx_hbm.at[pl.ds(wid * _NCHUNK, _NCHUNK)], idx_v)
        copies = [
            pltpu.async_copy(table.at[idx_v.at[j]],
                             rows_v.at[pl.ds(j * _CHUNK, _CHUNK)], sem)
            for j in range(_NCHUNK)
        ]
        for c in copies:
            c.wait()
        pltpu.sync_copy(rows_v, out_hbm.at[pl.ds(base, _RPW)])


_sc_gather = functools.partial(
    pl.kernel,
    out_type=(jax.ShapeDtypeStruct((_B, _H), jnp.float32),) * 3,
    mesh=plsc.VectorSubcoreMesh(core_axis_name="c", subcore_axis_name="s",
                                num_cores=_NC, num_subcores=_NS),
    scratch_types=[
        pltpu.VMEM((_NCHUNK, _CHUNK), jnp.int32),
        pltpu.VMEM((_RPW, _H), jnp.float32),
        pltpu.SemaphoreType.DMA,
    ],
)(_sc_gather_body)


_BS = 2048


def _tc_body(x_ref, y_ref, es_ref, ew_ref, el_ref,
             w1_ref, b1_ref, w2_ref, b2_ref, w3_ref, b3_ref, o_ref):
    h1 = jnp.maximum(
        x_ref[...] * w1_ref[0:1, :] + y_ref[...] * w1_ref[1:2, :] + b1_ref[...],
        0.0)
    h = jnp.dot(h1, w2_ref[...], preferred_element_type=jnp.float32) + b2_ref[...]
    acc = jnp.dot(es_ref[...], w3_ref[0:_H, :], preferred_element_type=jnp.float32)
    acc += jnp.dot(ew_ref[...], w3_ref[_H:2 * _H, :], preferred_element_type=jnp.float32)
    acc += jnp.dot(el_ref[...], w3_ref[2 * _H:3 * _H, :], preferred_element_type=jnp.float32)
    acc += jnp.dot(h, w3_ref[3 * _H:4 * _H, :], preferred_element_type=jnp.float32)
    o_ref[...] = jnp.maximum(acc + b3_ref[...], 0.0)


def _tc_project(x, y, es, ew, el, W1, b1, W2, b2, W3, b3):
    batch = pl.BlockSpec((_BS, _H), lambda i: (i, 0))
    col = pl.BlockSpec((_BS, 1), lambda i: (i, 0))
    full = lambda s: pl.BlockSpec(s, lambda i: (0, 0))
    return pl.pallas_call(
        _tc_body,
        grid=(_B // _BS,),
        in_specs=[col, col, batch, batch, batch,
                  full((2, _H)), full((1, _H)), full((_H, _H)),
                  full((1, _H)), full((4 * _H, _H)), full((1, _H))],
        out_specs=batch,
        out_shape=jax.ShapeDtypeStruct((_B, _H), jnp.float32),
    )(x, y, es, ew, el, W1, b1, W2, b2, W3, b3)


def kernel(spas_item_id, wl_id, wf_loc_id, wf_loc_x, wf_loc_y,
           spas_table, wl_table, loc_table, W1, b1, W2, b2, W3, b3):
    si = spas_item_id.astype(jnp.int32).reshape(_B // _CHUNK, _CHUNK)
    wi = wl_id.astype(jnp.int32).reshape(_B // _CHUNK, _CHUNK)
    li = wf_loc_id.astype(jnp.int32).reshape(_B // _CHUNK, _CHUNK)
    es, ew, el = _sc_gather(spas_table, wl_table, loc_table, si, wi, li)
    return _tc_project(wf_loc_x[:, None], wf_loc_y[:, None], es, ew, el,
                       W1, b1[None, :], W2, b2[None, :], W3, b3[None, :])
